# baseline (speedup 1.0000x reference)
import functools

import jax
import jax.numpy as jnp
from jax import lax
from jax.experimental import pallas as pl
from jax.experimental.pallas import tpu as pltpu
from jax.experimental.pallas import tpu_sc as plsc

NC = 2
NS = 16
NW = NC * NS
LANES = 16


def _make_kernel(B, L, V, D):
    R = B * L
    assert R % NW == 0
    RPW = R // NW
    CH = 192
    assert CH % LANES == 0 and CH % L == 0 and RPW % (2 * CH) == 0
    NCH = RPW // CH
    ND = D // LANES

    mesh = plsc.VectorSubcoreMesh(core_axis_name="c", subcore_axis_name="s")

    @functools.partial(
        pl.kernel,
        mesh=mesh,
        out_type=jax.ShapeDtypeStruct((R * D,), jnp.float32),
        scratch_types=[
            pltpu.VMEM((V, D), jnp.float32),
            pltpu.VMEM((L, D), jnp.float32),
            pltpu.VMEM((CH,), jnp.int32),
            pltpu.VMEM((CH,), jnp.int32),
            pltpu.VMEM((CH * D,), jnp.float32),
            pltpu.VMEM((CH * D,), jnp.float32),
            pltpu.SemaphoreType.DMA,
            pltpu.SemaphoreType.DMA,
            pltpu.SemaphoreType.DMA,
            pltpu.SemaphoreType.DMA,
        ],
    )
    def k(x_hbm, word_hbm, pos_hbm, out_hbm,
          word_v, pos_v, xb0, xb1, stage0, stage1, sw0, sw1, sx0, sx1):
        cid = lax.axis_index("c")
        sid = lax.axis_index("s")
        wid = sid * NC + cid
        base = wid * RPW

        pltpu.sync_copy(word_hbm, word_v)
        pltpu.sync_copy(pos_hbm, pos_v)

        def x_copy(ci, xb, sem):
            return pltpu.make_async_copy(
                x_hbm.at[pl.ds(base + ci * CH, CH)], xb, sem)

        def w_copy(ci, buf, sem):
            off = (base + ci * CH) * D
            return pltpu.make_async_copy(
                buf, out_hbm.at[pl.ds(off, CH * D)], sem)

        def expand(buf, xb):
            xs = [xb[pl.ds(t * LANES, LANES)] for t in range(CH // LANES)]

            def xget(i):
                return xs[i // LANES][i % LANES]

            def per_d(dd, carry):
                doff = pl.multiple_of(dd * LANES, LANES)
                sl = pl.ds(doff, LANES)
                pos_d = [pos_v[l, sl] for l in range(L)]
                prev = word_v[xget(0), sl]
                for i in range(1, CH + 1):
                    cur = word_v[xget(i), sl] if i < CH else None
                    r = i - 1
                    dst = pl.ds(pl.multiple_of(r * D + doff, LANES), LANES)
                    buf[dst] = prev + pos_d[r % L]
                    prev = cur
                return carry

            lax.fori_loop(0, ND, per_d, 0)

        x_copy(0, xb0, sx0).start()

        def pipe(g, carry):
            c0 = 2 * g
            c1 = 2 * g + 1

            @pl.when(g > 0)
            def _():
                w_copy(c0 - 2, stage0, sw0).wait()

            x_copy(c1, xb1, sx1).start()
            x_copy(c0, xb0, sx0).wait()
            expand(stage0, xb0)
            w_copy(c0, stage0, sw0).start()

            @pl.when(g > 0)
            def _():
                w_copy(c1 - 2, stage1, sw1).wait()

            @pl.when(g < NCH // 2 - 1)
            def _():
                x_copy(c0 + 2, xb0, sx0).start()

            x_copy(c1, xb1, sx1).wait()
            expand(stage1, xb1)
            w_copy(c1, stage1, sw1).start()
            return carry

        lax.fori_loop(0, NCH // 2, pipe, 0)
        w_copy(NCH - 2, stage0, sw0).wait()
        w_copy(NCH - 1, stage1, sw1).wait()

    return k


def kernel(X, word_embedding, pos_embedding):
    B, L = X.shape
    V, D = word_embedding.shape
    k = _make_kernel(B, L, V, D)
    x_flat = X.reshape(-1).astype(jnp.int32)
    return k(x_flat, word_embedding, pos_embedding).reshape(B, L, D)


# speedup vs baseline: 2.2262x; 2.2262x over previous
import functools

import jax
import jax.numpy as jnp
from jax import lax
from jax.experimental import pallas as pl
from jax.experimental.pallas import tpu as pltpu
from jax.experimental.pallas import tpu_sc as plsc

NC = 2
NS = 16
NW = NC * NS
LANES = 16
LP = 16


def _make_kernel(B, L, V, D):
    R = B * L
    assert R % NW == 0
    RPW = R // NW
    BPW = B // NW
    CB = 8
    CH = CB * L
    assert CH % LANES == 0 and BPW % (2 * CB) == 0
    NCH = BPW // CB
    ND = D // LANES

    mesh = plsc.VectorSubcoreMesh(core_axis_name="c", subcore_axis_name="s")

    @functools.partial(
        pl.kernel,
        mesh=mesh,
        out_type=jax.ShapeDtypeStruct((B, LP, D), jnp.float32),
        scratch_types=[
            pltpu.VMEM((V, D), jnp.float32),
            pltpu.VMEM((L, D), jnp.float32),
            pltpu.VMEM((CH,), jnp.int32),
            pltpu.VMEM((CH,), jnp.int32),
            pltpu.SMEM((CH,), jnp.int32),
            pltpu.VMEM((CB, LP, D), jnp.float32),
            pltpu.VMEM((CB, LP, D), jnp.float32),
            pltpu.SemaphoreType.DMA,
            pltpu.SemaphoreType.DMA,
            pltpu.SemaphoreType.DMA,
            pltpu.SemaphoreType.DMA,
        ],
    )
    def k(x_hbm, word_hbm, pos_hbm, out_hbm,
          word_v, pos_v, xb0, xb1, xsm, stage0, stage1, sw0, sw1, sx0, sx1):
        cid = lax.axis_index("c")
        sid = lax.axis_index("s")
        wid = sid * NC + cid
        base = wid * RPW
        bbase = wid * BPW

        pltpu.sync_copy(word_hbm, word_v)
        pltpu.sync_copy(pos_hbm, pos_v)

        sls = [pl.ds(d * LANES, LANES) for d in range(ND)]

        def x_copy(ci, xb, sem):
            return pltpu.make_async_copy(
                x_hbm.at[pl.ds(base + ci * CH, CH)], xb, sem)

        def w_copy(ci, buf, sem):
            return pltpu.make_async_copy(
                buf, out_hbm.at[pl.ds(bbase + ci * CB, CB)], sem)

        def expand(buf, xb):
            for t in range(CH // LANES):
                vec = xb[pl.ds(t * LANES, LANES)]
                for j in range(LANES):
                    xsm[t * LANES + j] = vec[j]

            def per_l(l, carry):
                pos_l = [pos_v[l, sl] for sl in sls]
                prev = [word_v[xsm[l], sl] for sl in sls]
                for kk in range(1, CB + 1):
                    cur = ([word_v[xsm[l + L * kk], sl] for sl in sls]
                           if kk < CB else None)
                    for d in range(ND):
                        buf[kk - 1, l, sls[d]] = prev[d] + pos_l[d]
                    prev = cur
                return carry

            lax.fori_loop(0, L, per_l, 0)

        x_copy(0, xb0, sx0).start()

        def pipe(g, carry):
            c0 = 2 * g
            c1 = 2 * g + 1

            @pl.when(g > 0)
            def _():
                w_copy(c0 - 2, stage0, sw0).wait()

            x_copy(c1, xb1, sx1).start()
            x_copy(c0, xb0, sx0).wait()
            expand(stage0, xb0)
            w_copy(c0, stage0, sw0).start()

            @pl.when(g > 0)
            def _():
                w_copy(c1 - 2, stage1, sw1).wait()

            @pl.when(g < NCH // 2 - 1)
            def _():
                x_copy(c0 + 2, xb0, sx0).start()

            x_copy(c1, xb1, sx1).wait()
            expand(stage1, xb1)
            w_copy(c1, stage1, sw1).start()
            return carry

        lax.fori_loop(0, NCH // 2, pipe, 0)
        w_copy(NCH - 2, stage0, sw0).wait()
        w_copy(NCH - 1, stage1, sw1).wait()

    return k


def kernel(X, word_embedding, pos_embedding):
    B, L = X.shape
    V, D = word_embedding.shape
    k = _make_kernel(B, L, V, D)
    x_flat = X.reshape(-1).astype(jnp.int32)
    return k(x_flat, word_embedding, pos_embedding)[:, :L, :]
